# baseline (speedup 1.0000x reference)
import functools

import jax
import jax.numpy as jnp
from jax import lax
from jax.experimental import pallas as pl
from jax.experimental.pallas import tpu as pltpu
from jax.experimental.pallas import tpu_sc as plsc

N_NODES = 10000
HALF = 5120
D_IN = 256
N_EDGES = 160000
EDGES_PER_TILE = 10240
N_EDGES_PAD = 16 * EDGES_PER_TILE
FCHUNK = 512
N_PIECES = EDGES_PER_TILE // FCHUNK
PCHUNK = 32
LIST_CAP = EDGES_PER_TILE + PCHUNK
STRIPE = HALF // 16

_MESH = dict(core_axis_name="c", subcore_axis_name="s",
             num_cores=2, num_subcores=16)


def _filter_body(src_hbm, dst_hbm, attr_hbm,
                 slist_hbm, wlist_hbm, cnt_hbm,
                 spb0, spb1, dpb0, dpb1, apb0, apb1,
                 slist, wlist, cbuf,
                 isem0, isem1):
    spb = [spb0, spb1]
    dpb = [dpb0, dpb1]
    apb = [apb0, apb1]
    isem = [isem0, isem1]

    c = lax.axis_index("c")
    s = lax.axis_index("s")
    lo = c * HALF
    hi = lo + HALF
    ebase = s * EDGES_PER_TILE

    def fire_piece(b, p):
        o = ebase + p * FCHUNK
        pltpu.async_copy(src_hbm.at[pl.ds(o, FCHUNK)], spb[b], isem[b])
        pltpu.async_copy(dst_hbm.at[pl.ds(o, FCHUNK)], dpb[b], isem[b])
        pltpu.async_copy(attr_hbm.at[pl.ds(o, FCHUNK)], apb[b], isem[b])

    def wait_piece(b, p):
        o = ebase + p * FCHUNK
        pltpu.make_async_copy(src_hbm.at[pl.ds(o, FCHUNK)], spb[b],
                              isem[b]).wait()
        pltpu.make_async_copy(dst_hbm.at[pl.ds(o, FCHUNK)], dpb[b],
                              isem[b]).wait()
        pltpu.make_async_copy(attr_hbm.at[pl.ds(o, FCHUNK)], apb[b],
                              isem[b]).wait()

    def filter_piece(b, cnt0):
        def grp(g, cnt):
            sl = pl.ds(16 * g, 16)
            d16 = dpb[b][sl]
            m = (d16 >= lo) & (d16 < hi)
            csum = plsc.cumsum(jnp.where(m, 1, 0))
            pos = csum + (cnt - 1)
            w16 = lax.shift_left(spb[b][sl], 13) | (d16 - lo)
            plsc.store_scatter(slist, [pos], w16, mask=m)
            plsc.store_scatter(wlist, [pos], apb[b][sl], mask=m)
            return cnt + csum[15]

        return lax.fori_loop(0, FCHUNK // 16, grp, cnt0)

    fire_piece(0, 0)

    def piece_pair(p, cnt):
        for u in range(2):
            pp = 2 * p + u
            wait_piece(u, pp)

            @pl.when(pp + 1 < N_PIECES)
            def _():
                fire_piece(1 - u, pp + 1)

            cnt = filter_piece(u, cnt)
        return cnt

    cnt = lax.fori_loop(0, N_PIECES // 2, piece_pair, jnp.int32(0))

    zpad = jnp.zeros((16,), jnp.int32)
    zpadf = jnp.zeros((16,), jnp.float32)
    for k in range(PCHUNK // 16):
        slist[pl.ds(cnt + 16 * k, 16)] = zpad
        wlist[pl.ds(cnt + 16 * k, 16)] = zpadf
    cbuf[pl.ds(0, 16)] = jnp.full((16,), cnt, jnp.int32)

    pltpu.sync_copy(slist, slist_hbm.at[c, s])
    pltpu.sync_copy(wlist, wlist_hbm.at[c, s])
    pltpu.sync_copy(cbuf, cnt_hbm.at[c, s])


@jax.jit
def _sc_filter(src, dst, attr):
    f = pl.kernel(
        _filter_body,
        out_type=(
            jax.ShapeDtypeStruct((2, 16, LIST_CAP), jnp.int32),
            jax.ShapeDtypeStruct((2, 16, LIST_CAP), jnp.float32),
            jax.ShapeDtypeStruct((2, 16, 16), jnp.int32),
        ),
        mesh=plsc.VectorSubcoreMesh(**_MESH),
        scratch_types=(
            [pltpu.VMEM((FCHUNK,), jnp.int32) for _ in range(4)]
            + [pltpu.VMEM((FCHUNK,), jnp.float32) for _ in range(2)]
            + [pltpu.VMEM((LIST_CAP,), jnp.int32)]
            + [pltpu.VMEM((LIST_CAP,), jnp.float32)]
            + [pltpu.VMEM((16,), jnp.int32)]
            + [pltpu.SemaphoreType.DMA for _ in range(2)]
        ),
        compiler_params=pltpu.CompilerParams(needs_layout_passes=False),
        name="gnn_edge_filter_sc",
    )
    return f(src, dst, attr)


def _pipe_body(x_hbm, slist_hbm, wlist_hbm, cnt_hbm, out_hbm,
               spc0, spc1, wpc0, wpc1, cbuf,
               gidx0, gidx1, dstl0, dstl1,
               rows0, rows1, rlo0, rlo1, rhi0, rhi1,
               agg_lo, agg_hi,
               lsem, psem0, psem1, gsem0, gsem1, ssem0, ssem1):
    spc = [spc0, spc1]
    wpc = [wpc0, wpc1]
    gidx = [gidx0, gidx1]
    dstl = [dstl0, dstl1]
    rows = [rows0, rows1]
    rlo = [rlo0, rlo1]
    rhi = [rhi0, rhi1]
    psem = [psem0, psem1]
    gsem = [gsem0, gsem1]
    ssem = [ssem0, ssem1]

    c = lax.axis_index("c")
    s = lax.axis_index("s")
    stripe_base = s * STRIPE

    pltpu.async_copy(cnt_hbm.at[c, s], cbuf, lsem)

    def zero_rows(i, carry):
        for f in range(128 // 16):
            rlo0[i, pl.ds(16 * f, 16)] = jnp.zeros((16,), jnp.float32)
        return carry

    lax.fori_loop(0, PCHUNK, zero_rows, 0)

    def zero_stripe(r, carry):
        sl = pl.ds(stripe_base + r * PCHUNK, PCHUNK)
        pltpu.sync_copy(rlo0, agg_lo.at[sl])
        pltpu.sync_copy(rlo0, agg_hi.at[sl])
        return carry

    lax.fori_loop(0, STRIPE // PCHUNK, zero_stripe, 0)

    pltpu.make_async_copy(cnt_hbm.at[c, s], cbuf, lsem).wait()
    cnt = cbuf[pl.ds(0, 16)][0]
    nch = (cnt + PCHUNK - 1) // PCHUNK
    plsc.subcore_barrier()

    def fire_piece(b, q):
        o = q * PCHUNK
        pltpu.async_copy(slist_hbm.at[c, s, pl.ds(o, PCHUNK)], spc[b],
                         psem[b])
        pltpu.async_copy(wlist_hbm.at[c, s, pl.ds(o, PCHUNK)], wpc[b],
                         psem[b])

    def wait_piece(b, q):
        o = q * PCHUNK
        pltpu.make_async_copy(slist_hbm.at[c, s, pl.ds(o, PCHUNK)], spc[b],
                              psem[b]).wait()
        pltpu.make_async_copy(wlist_hbm.at[c, s, pl.ds(o, PCHUNK)], wpc[b],
                              psem[b]).wait()

    def build_idx(b):
        for k in range(PCHUNK // 16):
            w16 = spc[b][pl.ds(16 * k, 16)]
            gidx[b][pl.ds(16 * k, 16)] = lax.shift_right_logical(w16, 13)
            dstl[b][pl.ds(16 * k, 16)] = w16 & (8192 - 1)

    def scale_buf(b):
        rb = rows[b]
        lb = rlo[b]
        hb = rhi[b]
        ab = wpc[b]

        def grp(g, carry):
            avec = ab[pl.ds(16 * g, 16)]
            for l in range(16):
                a = avec[l]
                row = 16 * g + l
                for f in range(8):
                    lb[row, pl.ds(16 * f, 16)] = rb[row, pl.ds(16 * f, 16)] * a
                for f in range(8):
                    hb[row, pl.ds(16 * f, 16)] = (
                        rb[row, pl.ds(128 + 16 * f, 16)] * a
                    )
            return carry

        lax.fori_loop(0, PCHUNK // 16, grp, 0)

    def fire_scatter(b):
        pltpu.async_copy(rlo[b], agg_lo.at[dstl[b]], ssem[b], add=True)
        pltpu.async_copy(rhi[b], agg_hi.at[dstl[b]], ssem[b], add=True)

    def drain_scatter(b):
        pltpu.make_async_copy(rlo[b], agg_lo.at[dstl[b]], ssem[b]).wait()
        pltpu.make_async_copy(rhi[b], agg_hi.at[dstl[b]], ssem[b]).wait()

    @pl.when(nch > 0)
    def _():
        fire_piece(0, 0)
        fire_piece(1, 1)
        wait_piece(0, 0)
        build_idx(0)
        pltpu.async_copy(x_hbm.at[gidx[0]], rows[0], gsem[0])

    def chunk_pair(p, carry):
        for u in range(2):
            q = 2 * p + u
            b = u
            ob = 1 - u

            @pl.when(q < nch)
            def _():
                pltpu.make_async_copy(x_hbm.at[gidx[b]], rows[b],
                                      gsem[b]).wait()
                scale_buf(b)

                @pl.when(q >= 1)
                def _():
                    drain_scatter(ob)

                @pl.when(q + 2 < nch)
                def _():
                    fire_piece(b, q + 2)

                @pl.when(q + 1 < nch)
                def _():
                    wait_piece(ob, q + 1)
                    build_idx(ob)
                    pltpu.async_copy(x_hbm.at[gidx[ob]], rows[ob], gsem[ob])

                fire_scatter(b)

        return carry

    lax.fori_loop(0, (nch + 1) // 2, chunk_pair, 0)

    @pl.when((nch > 0) & (lax.rem(nch, 2) == 1))
    def _():
        drain_scatter(0)

    @pl.when((nch > 0) & (lax.rem(nch, 2) == 0))
    def _():
        drain_scatter(1)

    plsc.subcore_barrier()

    pltpu.sync_copy(agg_lo.at[pl.ds(stripe_base, STRIPE)],
                    out_hbm.at[c, 0, pl.ds(stripe_base, STRIPE)])
    pltpu.sync_copy(agg_hi.at[pl.ds(stripe_base, STRIPE)],
                    out_hbm.at[c, 1, pl.ds(stripe_base, STRIPE)])


@jax.jit
def _sc_pipeline(x, slists, wlists, cnts):
    f = pl.kernel(
        _pipe_body,
        out_type=jax.ShapeDtypeStruct((2, 2, HALF, 128), jnp.float32),
        mesh=plsc.VectorSubcoreMesh(**_MESH),
        scratch_types=(
            [pltpu.VMEM((PCHUNK,), jnp.int32) for _ in range(2)]
            + [pltpu.VMEM((PCHUNK,), jnp.float32) for _ in range(2)]
            + [pltpu.VMEM((16,), jnp.int32)]
            + [pltpu.VMEM((PCHUNK,), jnp.int32) for _ in range(4)]
            + [pltpu.VMEM((PCHUNK, D_IN), jnp.float32) for _ in range(2)]
            + [pltpu.VMEM((PCHUNK, 128), jnp.float32) for _ in range(4)]
            + [pltpu.VMEM_SHARED((HALF, 128), jnp.float32) for _ in range(2)]
            + [pltpu.SemaphoreType.DMA for _ in range(7)]
        ),
        name="gnn_segment_sum_sc",
    )
    return f(x, slists, wlists, cnts)


def _tc_body(agg_ref, x_ref, wrel_ref, wroot_ref, wfc_ref, brel_ref,
             bfc_ref, out_ref):
    h = jnp.dot(agg_ref[...], wrel_ref[...],
                preferred_element_type=jnp.float32)
    h += jnp.dot(x_ref[...], wroot_ref[...],
                 preferred_element_type=jnp.float32)
    h += brel_ref[...]
    h = jnp.maximum(h, 0.0)
    out_ref[...] = (
        jnp.dot(h, wfc_ref[...], preferred_element_type=jnp.float32)
        + bfc_ref[...]
    )


@functools.partial(jax.jit, static_argnames=())
def _tc_dense(agg, x, wrelT, wrootT, wfcT, brel, bfc):
    n, d_in = x.shape
    d_hid = wrootT.shape[1]
    n_cls = wfcT.shape[1]
    blk = 1000
    grid = (n // blk,)
    return pl.pallas_call(
        _tc_body,
        grid=grid,
        in_specs=[
            pl.BlockSpec((blk, d_in), lambda i: (i, 0)),
            pl.BlockSpec((blk, d_in), lambda i: (i, 0)),
            pl.BlockSpec((d_in, d_hid), lambda i: (0, 0)),
            pl.BlockSpec((d_in, d_hid), lambda i: (0, 0)),
            pl.BlockSpec((d_hid, n_cls), lambda i: (0, 0)),
            pl.BlockSpec((1, d_hid), lambda i: (0, 0)),
            pl.BlockSpec((1, n_cls), lambda i: (0, 0)),
        ],
        out_specs=pl.BlockSpec((blk, n_cls), lambda i: (i, 0)),
        out_shape=jax.ShapeDtypeStruct((n, n_cls), jnp.float32),
    )(agg, x, wrelT, wrootT, wfcT, brel, bfc)


def kernel(x, edge_index, edge_attr, W_rel, b_rel, W_root, W_fc, b_fc):
    src = edge_index[0]
    dst = edge_index[1]
    pad = N_EDGES_PAD - src.shape[0]
    src = jnp.concatenate([src, jnp.zeros((pad,), src.dtype)])
    dst = jnp.concatenate([dst, jnp.full((pad,), N_NODES, dst.dtype)])
    attr = jnp.concatenate([edge_attr, jnp.zeros((pad,), edge_attr.dtype)])
    slists, wlists, cnts = _sc_filter(src, dst, attr)
    aggh = _sc_pipeline(x, slists, wlists, cnts)
    aggc = jnp.concatenate([aggh[:, 0], aggh[:, 1]], axis=-1)
    agg = jnp.concatenate([aggc[0], aggc[1][: N_NODES - HALF]], axis=0)
    out = _tc_dense(agg, x, W_rel.T, W_root.T, W_fc.T,
                    b_rel[None, :], b_fc[None, :])
    return out


# speedup vs baseline: 1.0006x; 1.0006x over previous
import functools

import jax
import jax.numpy as jnp
from jax import lax
from jax.experimental import pallas as pl
from jax.experimental.pallas import tpu as pltpu
from jax.experimental.pallas import tpu_sc as plsc

N_NODES = 10000
HALF = 5120
D_IN = 256
N_EDGES = 160000
EDGES_PER_TILE = 10240
N_EDGES_PAD = 16 * EDGES_PER_TILE
FCHUNK = 512
N_PIECES = EDGES_PER_TILE // FCHUNK
PCHUNK = 32
LIST_CAP = EDGES_PER_TILE + PCHUNK
STRIPE = HALF // 16

_MESH = dict(core_axis_name="c", subcore_axis_name="s",
             num_cores=2, num_subcores=16)


def _filter_body(src_hbm, dst_hbm, attr_hbm,
                 slist_hbm, wlist_hbm, cnt_hbm,
                 spb0, spb1, dpb0, dpb1, apb0, apb1,
                 slist, wlist, cbuf,
                 isem0, isem1):
    spb = [spb0, spb1]
    dpb = [dpb0, dpb1]
    apb = [apb0, apb1]
    isem = [isem0, isem1]

    c = lax.axis_index("c")
    s = lax.axis_index("s")
    lo = c * HALF
    hi = lo + HALF
    ebase = s * EDGES_PER_TILE

    def fire_piece(b, p):
        o = ebase + p * FCHUNK
        pltpu.async_copy(src_hbm.at[pl.ds(o, FCHUNK)], spb[b], isem[b])
        pltpu.async_copy(dst_hbm.at[pl.ds(o, FCHUNK)], dpb[b], isem[b])
        pltpu.async_copy(attr_hbm.at[pl.ds(o, FCHUNK)], apb[b], isem[b])

    def wait_piece(b, p):
        o = ebase + p * FCHUNK
        pltpu.make_async_copy(src_hbm.at[pl.ds(o, FCHUNK)], spb[b],
                              isem[b]).wait()
        pltpu.make_async_copy(dst_hbm.at[pl.ds(o, FCHUNK)], dpb[b],
                              isem[b]).wait()
        pltpu.make_async_copy(attr_hbm.at[pl.ds(o, FCHUNK)], apb[b],
                              isem[b]).wait()

    def filter_piece(b, cnt0):
        def grp(g, cnt):
            sl = pl.ds(16 * g, 16)
            d16 = dpb[b][sl]
            m = (d16 >= lo) & (d16 < hi)
            csum = plsc.cumsum(jnp.where(m, 1, 0))
            pos = csum + (cnt - 1)
            w16 = lax.shift_left(spb[b][sl], 13) | (d16 - lo)
            plsc.store_scatter(slist, [pos], w16, mask=m)
            plsc.store_scatter(wlist, [pos], apb[b][sl], mask=m)
            return cnt + csum[15]

        return lax.fori_loop(0, FCHUNK // 16, grp, cnt0)

    fire_piece(0, 0)

    def piece_pair(p, cnt):
        for u in range(2):
            pp = 2 * p + u
            wait_piece(u, pp)

            @pl.when(pp + 1 < N_PIECES)
            def _():
                fire_piece(1 - u, pp + 1)

            cnt = filter_piece(u, cnt)
        return cnt

    cnt = lax.fori_loop(0, N_PIECES // 2, piece_pair, jnp.int32(0))

    zpadf = jnp.zeros((16,), jnp.float32)
    for k in range(PCHUNK // 16):
        slist[pl.ds(cnt + 16 * k, 16)] = (
            jax.lax.iota(jnp.int32, 16) + 16 * k
        )
        wlist[pl.ds(cnt + 16 * k, 16)] = zpadf
    cbuf[pl.ds(0, 16)] = jnp.full((16,), cnt, jnp.int32)

    pltpu.sync_copy(slist, slist_hbm.at[c, s])
    pltpu.sync_copy(wlist, wlist_hbm.at[c, s])
    pltpu.sync_copy(cbuf, cnt_hbm.at[c, s])


@jax.jit
def _sc_filter(src, dst, attr):
    f = pl.kernel(
        _filter_body,
        out_type=(
            jax.ShapeDtypeStruct((2, 16, LIST_CAP), jnp.int32),
            jax.ShapeDtypeStruct((2, 16, LIST_CAP), jnp.float32),
            jax.ShapeDtypeStruct((2, 16, 16), jnp.int32),
        ),
        mesh=plsc.VectorSubcoreMesh(**_MESH),
        scratch_types=(
            [pltpu.VMEM((FCHUNK,), jnp.int32) for _ in range(4)]
            + [pltpu.VMEM((FCHUNK,), jnp.float32) for _ in range(2)]
            + [pltpu.VMEM((LIST_CAP,), jnp.int32)]
            + [pltpu.VMEM((LIST_CAP,), jnp.float32)]
            + [pltpu.VMEM((16,), jnp.int32)]
            + [pltpu.SemaphoreType.DMA for _ in range(2)]
        ),
        compiler_params=pltpu.CompilerParams(needs_layout_passes=False),
        name="gnn_edge_filter_sc",
    )
    return f(src, dst, attr)


def _pipe_body(x_hbm, slist_hbm, wlist_hbm, cnt_hbm, out_hbm,
               spc0, spc1, wpc0, wpc1, cbuf,
               gidx0, gidx1, dstl0, dstl1,
               rows0, rows1, rlo0, rlo1, rhi0, rhi1,
               agg_lo, agg_hi,
               lsem, psem0, psem1, gsem0, gsem1, ssem0, ssem1):
    spc = [spc0, spc1]
    wpc = [wpc0, wpc1]
    gidx = [gidx0, gidx1]
    dstl = [dstl0, dstl1]
    rows = [rows0, rows1]
    rlo = [rlo0, rlo1]
    rhi = [rhi0, rhi1]
    psem = [psem0, psem1]
    gsem = [gsem0, gsem1]
    ssem = [ssem0, ssem1]

    c = lax.axis_index("c")
    s = lax.axis_index("s")
    stripe_base = s * STRIPE

    pltpu.async_copy(cnt_hbm.at[c, s], cbuf, lsem)

    def zero_rows(i, carry):
        for f in range(128 // 16):
            rlo0[i, pl.ds(16 * f, 16)] = jnp.zeros((16,), jnp.float32)
        return carry

    lax.fori_loop(0, PCHUNK, zero_rows, 0)

    def zero_stripe(r, carry):
        sl = pl.ds(stripe_base + r * PCHUNK, PCHUNK)
        pltpu.sync_copy(rlo0, agg_lo.at[sl])
        pltpu.sync_copy(rlo0, agg_hi.at[sl])
        return carry

    lax.fori_loop(0, STRIPE // PCHUNK, zero_stripe, 0)

    pltpu.make_async_copy(cnt_hbm.at[c, s], cbuf, lsem).wait()
    cnt = cbuf[pl.ds(0, 16)][0]
    nch = (cnt + PCHUNK - 1) // PCHUNK
    plsc.subcore_barrier()

    def fire_piece(b, q):
        o = q * PCHUNK
        pltpu.async_copy(slist_hbm.at[c, s, pl.ds(o, PCHUNK)], spc[b],
                         psem[b])
        pltpu.async_copy(wlist_hbm.at[c, s, pl.ds(o, PCHUNK)], wpc[b],
                         psem[b])

    def wait_piece(b, q):
        o = q * PCHUNK
        pltpu.make_async_copy(slist_hbm.at[c, s, pl.ds(o, PCHUNK)], spc[b],
                              psem[b]).wait()
        pltpu.make_async_copy(wlist_hbm.at[c, s, pl.ds(o, PCHUNK)], wpc[b],
                              psem[b]).wait()

    def build_idx(b):
        for k in range(PCHUNK // 16):
            w16 = spc[b][pl.ds(16 * k, 16)]
            gidx[b][pl.ds(16 * k, 16)] = lax.shift_right_logical(w16, 13)
            dstl[b][pl.ds(16 * k, 16)] = w16 & (8192 - 1)

    def scale_buf(b):
        rb = rows[b]
        lb = rlo[b]
        hb = rhi[b]
        ab = wpc[b]

        def grp(g, carry):
            avec = ab[pl.ds(16 * g, 16)]
            for l in range(16):
                a = avec[l]
                row = 16 * g + l
                for f in range(8):
                    lb[row, pl.ds(16 * f, 16)] = rb[row, pl.ds(16 * f, 16)] * a
                for f in range(8):
                    hb[row, pl.ds(16 * f, 16)] = (
                        rb[row, pl.ds(128 + 16 * f, 16)] * a
                    )
            return carry

        lax.fori_loop(0, PCHUNK // 16, grp, 0)

    def fire_scatter(b):
        pltpu.async_copy(rlo[b], agg_lo.at[dstl[b]], ssem[b], add=True)
        pltpu.async_copy(rhi[b], agg_hi.at[dstl[b]], ssem[b], add=True)

    def drain_scatter(b):
        pltpu.make_async_copy(rlo[b], agg_lo.at[dstl[b]], ssem[b]).wait()
        pltpu.make_async_copy(rhi[b], agg_hi.at[dstl[b]], ssem[b]).wait()

    @pl.when(nch > 0)
    def _():
        fire_piece(0, 0)
        fire_piece(1, 1)
        wait_piece(0, 0)
        build_idx(0)
        pltpu.async_copy(x_hbm.at[gidx[0]], rows[0], gsem[0])

    def chunk_pair(p, carry):
        for u in range(2):
            q = 2 * p + u
            b = u
            ob = 1 - u

            @pl.when(q < nch)
            def _():
                pltpu.make_async_copy(x_hbm.at[gidx[b]], rows[b],
                                      gsem[b]).wait()
                scale_buf(b)

                @pl.when(q >= 1)
                def _():
                    drain_scatter(ob)

                @pl.when(q + 2 < nch)
                def _():
                    fire_piece(b, q + 2)

                @pl.when(q + 1 < nch)
                def _():
                    wait_piece(ob, q + 1)
                    build_idx(ob)
                    pltpu.async_copy(x_hbm.at[gidx[ob]], rows[ob], gsem[ob])

                fire_scatter(b)

        return carry

    lax.fori_loop(0, (nch + 1) // 2, chunk_pair, 0)

    @pl.when((nch > 0) & (lax.rem(nch, 2) == 1))
    def _():
        drain_scatter(0)

    @pl.when((nch > 0) & (lax.rem(nch, 2) == 0))
    def _():
        drain_scatter(1)

    plsc.subcore_barrier()

    pltpu.sync_copy(agg_lo.at[pl.ds(stripe_base, STRIPE)],
                    out_hbm.at[c, 0, pl.ds(stripe_base, STRIPE)])
    pltpu.sync_copy(agg_hi.at[pl.ds(stripe_base, STRIPE)],
                    out_hbm.at[c, 1, pl.ds(stripe_base, STRIPE)])


@jax.jit
def _sc_pipeline(x, slists, wlists, cnts):
    f = pl.kernel(
        _pipe_body,
        out_type=jax.ShapeDtypeStruct((2, 2, HALF, 128), jnp.float32),
        mesh=plsc.VectorSubcoreMesh(**_MESH),
        scratch_types=(
            [pltpu.VMEM((PCHUNK,), jnp.int32) for _ in range(2)]
            + [pltpu.VMEM((PCHUNK,), jnp.float32) for _ in range(2)]
            + [pltpu.VMEM((16,), jnp.int32)]
            + [pltpu.VMEM((PCHUNK,), jnp.int32) for _ in range(4)]
            + [pltpu.VMEM((PCHUNK, D_IN), jnp.float32) for _ in range(2)]
            + [pltpu.VMEM((PCHUNK, 128), jnp.float32) for _ in range(4)]
            + [pltpu.VMEM_SHARED((HALF, 128), jnp.float32) for _ in range(2)]
            + [pltpu.SemaphoreType.DMA for _ in range(7)]
        ),
        name="gnn_segment_sum_sc",
    )
    return f(x, slists, wlists, cnts)


def _tc_body(agg_ref, x_ref, wrel_ref, wroot_ref, wfc_ref, brel_ref,
             bfc_ref, out_ref):
    h = jnp.dot(agg_ref[...], wrel_ref[...],
                preferred_element_type=jnp.float32)
    h += jnp.dot(x_ref[...], wroot_ref[...],
                 preferred_element_type=jnp.float32)
    h += brel_ref[...]
    h = jnp.maximum(h, 0.0)
    out_ref[...] = (
        jnp.dot(h, wfc_ref[...], preferred_element_type=jnp.float32)
        + bfc_ref[...]
    )


@functools.partial(jax.jit, static_argnames=())
def _tc_dense(agg, x, wrelT, wrootT, wfcT, brel, bfc):
    n, d_in = x.shape
    d_hid = wrootT.shape[1]
    n_cls = wfcT.shape[1]
    blk = 1000
    grid = (n // blk,)
    return pl.pallas_call(
        _tc_body,
        grid=grid,
        in_specs=[
            pl.BlockSpec((blk, d_in), lambda i: (i, 0)),
            pl.BlockSpec((blk, d_in), lambda i: (i, 0)),
            pl.BlockSpec((d_in, d_hid), lambda i: (0, 0)),
            pl.BlockSpec((d_in, d_hid), lambda i: (0, 0)),
            pl.BlockSpec((d_hid, n_cls), lambda i: (0, 0)),
            pl.BlockSpec((1, d_hid), lambda i: (0, 0)),
            pl.BlockSpec((1, n_cls), lambda i: (0, 0)),
        ],
        out_specs=pl.BlockSpec((blk, n_cls), lambda i: (i, 0)),
        out_shape=jax.ShapeDtypeStruct((n, n_cls), jnp.float32),
    )(agg, x, wrelT, wrootT, wfcT, brel, bfc)


def kernel(x, edge_index, edge_attr, W_rel, b_rel, W_root, W_fc, b_fc):
    src = edge_index[0]
    dst = edge_index[1]
    pad = N_EDGES_PAD - src.shape[0]
    src = jnp.concatenate([src, jnp.zeros((pad,), src.dtype)])
    dst = jnp.concatenate(
        [dst, N_NODES + (jnp.arange(pad, dtype=dst.dtype) % 240)])
    attr = jnp.concatenate([edge_attr, jnp.zeros((pad,), edge_attr.dtype)])
    slists, wlists, cnts = _sc_filter(src, dst, attr)
    aggh = _sc_pipeline(x, slists, wlists, cnts)
    aggc = jnp.concatenate([aggh[:, 0], aggh[:, 1]], axis=-1)
    agg = jnp.concatenate([aggc[0], aggc[1][: N_NODES - HALF]], axis=0)
    out = _tc_dense(agg, x, W_rel.T, W_root.T, W_fc.T,
                    b_rel[None, :], b_fc[None, :])
    return out


# speedup vs baseline: 3.9349x; 3.9328x over previous
import functools

import jax
import jax.numpy as jnp
from jax import lax
from jax.experimental import pallas as pl
from jax.experimental.pallas import tpu as pltpu
from jax.experimental.pallas import tpu_sc as plsc

N_NODES = 10000
N_PAD = 10240
D_HALF = 128
N_EDGES = 160000
EDGES_PER_TILE = 10000
CHUNK = 80
N_CHUNKS = EDGES_PER_TILE // CHUNK
STRIPE = N_PAD // 16


NBUF = 4


def _sc_body(xs_hbm, src_hbm, dst_hbm, attr_hbm, out_hbm,
             sbuf0, sbuf1, sbuf2, sbuf3,
             gidx0, gidx1, gidx2, gidx3,
             dstb0, dstb1, dstb2, dstb3,
             abuf0, abuf1, abuf2, abuf3,
             rows0, rows1, rows2, rows3,
             agg_sh,
             gsem0, gsem1, gsem2, gsem3,
             ssem0, ssem1, ssem2, ssem3,
             isem0, isem1, isem2, isem3):
    sbuf = [sbuf0, sbuf1, sbuf2, sbuf3]
    gidx = [gidx0, gidx1, gidx2, gidx3]
    dstb = [dstb0, dstb1, dstb2, dstb3]
    abuf = [abuf0, abuf1, abuf2, abuf3]
    rows = [rows0, rows1, rows2, rows3]
    gsem = [gsem0, gsem1, gsem2, gsem3]
    ssem = [ssem0, ssem1, ssem2, ssem3]
    isem = [isem0, isem1, isem2, isem3]

    c = lax.axis_index("c")
    s = lax.axis_index("s")
    stripe_base = s * STRIPE
    ebase = s * EDGES_PER_TILE
    coff = c * N_NODES

    def fire_idx(b, cidx):
        o = ebase + cidx * CHUNK
        pltpu.async_copy(src_hbm.at[pl.ds(o, CHUNK)], sbuf[b], isem[b])
        pltpu.async_copy(dst_hbm.at[pl.ds(o, CHUNK)], dstb[b], isem[b])
        pltpu.async_copy(attr_hbm.at[pl.ds(o, CHUNK)], abuf[b], isem[b])

    def wait_idx(b, cidx):
        o = ebase + cidx * CHUNK
        pltpu.make_async_copy(src_hbm.at[pl.ds(o, CHUNK)], sbuf[b],
                              isem[b]).wait()
        pltpu.make_async_copy(dst_hbm.at[pl.ds(o, CHUNK)], dstb[b],
                              isem[b]).wait()
        pltpu.make_async_copy(attr_hbm.at[pl.ds(o, CHUNK)], abuf[b],
                              isem[b]).wait()

    def build_gidx(b):
        for v in range(CHUNK // 16):
            gidx[b][pl.ds(16 * v, 16)] = sbuf[b][pl.ds(16 * v, 16)] + coff

    def scale_buf(b):
        rb = rows[b]
        ab = abuf[b]

        def scale(g, carry2):
            avec = ab[pl.ds(g * 16, 16)]
            for l in range(16):
                a = avec[l]
                row = g * 16 + l
                for f in range(8):
                    rb[row, pl.ds(16 * f, 16)] = rb[row, pl.ds(16 * f, 16)] * a
            return carry2

        lax.fori_loop(0, CHUNK // 16, scale, 0)

    def zero_rows(i, carry):
        for f in range(8):
            rows0[i, pl.ds(16 * f, 16)] = jnp.zeros((16,), jnp.float32)
        return carry

    lax.fori_loop(0, CHUNK, zero_rows, 0)

    def zero_stripe(r, carry):
        pltpu.sync_copy(rows0, agg_sh.at[pl.ds(stripe_base + r * CHUNK, CHUNK)])
        return carry

    lax.fori_loop(0, STRIPE // CHUNK, zero_stripe, 0)
    plsc.subcore_barrier()

    fire_idx(0, 0)
    fire_idx(1, 1)
    wait_idx(0, 0)
    build_gidx(0)
    pltpu.async_copy(xs_hbm.at[gidx[0]], rows[0], gsem[0])

    def slot_group(t, carry):
        for u in range(NBUF):
            j = NBUF * t + u
            b = u
            b1 = (u + 1) % NBUF
            bq = (u + 2) % NBUF

            bp = (u + 3) % NBUF

            @pl.when(j < N_CHUNKS)
            def _():
                pltpu.make_async_copy(xs_hbm.at[gidx[b]], rows[b],
                                      gsem[b]).wait()
                scale_buf(b)

                @pl.when(j >= 1)
                def _():
                    pltpu.make_async_copy(rows[bp], agg_sh.at[dstb[bp]],
                                          ssem[bp]).wait()

                pltpu.async_copy(rows[b], agg_sh.at[dstb[b]], ssem[b],
                                 add=True)

                @pl.when(j + 2 < N_CHUNKS)
                def _():
                    fire_idx(bq, j + 2)

                @pl.when(j + 1 < N_CHUNKS)
                def _():
                    wait_idx(b1, j + 1)
                    build_gidx(b1)
                    pltpu.async_copy(xs_hbm.at[gidx[b1]], rows[b1], gsem[b1])

        return carry

    lax.fori_loop(0, (N_CHUNKS + NBUF - 1) // NBUF, slot_group, 0)

    b_last = (N_CHUNKS - 1) % NBUF
    pltpu.make_async_copy(rows[b_last], agg_sh.at[dstb[b_last]],
                          ssem[b_last]).wait()
    plsc.subcore_barrier()

    def writeback(r, carry):
        b = stripe_base + r * CHUNK
        pltpu.sync_copy(agg_sh.at[pl.ds(b, CHUNK)], out_hbm.at[c, pl.ds(b, CHUNK)])
        return carry

    lax.fori_loop(0, STRIPE // CHUNK, writeback, 0)


@jax.jit
def _sc_segment(xs, src, dst, attr):
    mesh = plsc.VectorSubcoreMesh(core_axis_name="c", subcore_axis_name="s",
                                  num_cores=2, num_subcores=16)
    f = pl.kernel(
        _sc_body,
        out_type=jax.ShapeDtypeStruct((2, N_PAD, D_HALF), jnp.float32),
        mesh=mesh,
        scratch_types=(
            [pltpu.VMEM((CHUNK,), jnp.int32) for _ in range(3 * NBUF)]
            + [pltpu.VMEM((CHUNK,), jnp.float32) for _ in range(NBUF)]
            + [pltpu.VMEM((CHUNK, D_HALF), jnp.float32) for _ in range(NBUF)]
            + [pltpu.VMEM_SHARED((N_PAD, D_HALF), jnp.float32)]
            + [pltpu.SemaphoreType.DMA for _ in range(3 * NBUF)]
        ),
        name="gnn_segment_sum_sc",
    )
    return f(xs, src, dst, attr)


def _tc_body(aggh_ref, x_ref, wrel_ref, wroot_ref, wfc_ref, brel_ref,
             bfc_ref, out_ref):
    a = aggh_ref[...]
    h = jnp.dot(a[0], wrel_ref[0], preferred_element_type=jnp.float32)
    h += jnp.dot(a[1], wrel_ref[1], preferred_element_type=jnp.float32)
    h += jnp.dot(x_ref[...], wroot_ref[...], preferred_element_type=jnp.float32)
    h += brel_ref[...]
    h = jnp.maximum(h, 0.0)
    out_ref[...] = (
        jnp.dot(h, wfc_ref[...], preferred_element_type=jnp.float32)
        + bfc_ref[...]
    )


@functools.partial(jax.jit, static_argnames=())
def _tc_dense(aggh, x, wrelT3, wrootT, wfcT, brel, bfc):
    n, d_in = x.shape
    d_hid = wrootT.shape[1]
    n_cls = wfcT.shape[1]
    blk = 1000
    grid = (n // blk,)
    return pl.pallas_call(
        _tc_body,
        grid=grid,
        in_specs=[
            pl.BlockSpec((2, blk, D_HALF), lambda i: (0, i, 0)),
            pl.BlockSpec((blk, d_in), lambda i: (i, 0)),
            pl.BlockSpec((2, D_HALF, d_hid), lambda i: (0, 0, 0)),
            pl.BlockSpec((d_in, d_hid), lambda i: (0, 0)),
            pl.BlockSpec((d_hid, n_cls), lambda i: (0, 0)),
            pl.BlockSpec((1, d_hid), lambda i: (0, 0)),
            pl.BlockSpec((1, n_cls), lambda i: (0, 0)),
        ],
        out_specs=pl.BlockSpec((blk, n_cls), lambda i: (i, 0)),
        out_shape=jax.ShapeDtypeStruct((n, n_cls), jnp.float32),
    )(aggh, x, wrelT3, wrootT, wfcT, brel, bfc)


def kernel(x, edge_index, edge_attr, W_rel, b_rel, W_root, W_fc, b_fc):
    src = edge_index[0]
    dst = edge_index[1]
    xs = jnp.concatenate([x[:, :D_HALF], x[:, D_HALF:]], axis=0)
    aggh = _sc_segment(xs, src, dst, edge_attr)
    wrelT3 = W_rel.T.reshape(2, D_HALF, -1)
    out = _tc_dense(aggh, x, wrelT3, W_root.T, W_fc.T,
                    b_rel[None, :], b_fc[None, :])
    return out
